# baseline (speedup 1.0000x reference)
import jax
import jax.numpy as jnp
from jax import lax
from jax.experimental import pallas as pl


def _roll(x, sh, axis):
    if axis == 0:
        return jnp.concatenate([x[-sh:, :], x[:-sh, :]], axis=0)
    return jnp.concatenate([x[:, -sh:], x[:, :-sh]], axis=1)


def _roll3(x, sh, axis):
    idx = [slice(None)] * 3
    lo = [slice(None)] * 3
    idx[axis] = slice(-sh, None)
    lo[axis] = slice(None, -sh)
    return jnp.concatenate([x[tuple(idx)], x[tuple(lo)]], axis=axis)


def _lovasz_body(p_ref, t_ref, o_ref):
    p = p_ref[0]
    t = t_ref[0]
    R, C = p.shape
    N = R * C
    LOGN = N.bit_length() - 1

    row = lax.broadcasted_iota(jnp.int32, (R, C), 0)
    lane = lax.broadcasted_iota(jnp.int32, (R, C), 1)

    signs = 2.0 * t - 1.0
    e = 1.0 - p * signs
    bits = lax.bitcast_convert_type(e, jnp.int32)
    bits = (bits & jnp.int32(-2)) | t.astype(jnp.int32)
    y = bits ^ ((bits >> 31) & jnp.int32(0x7FFFFFFF))
    z = ~y

    R8 = R // 8
    a0 = lax.broadcasted_iota(jnp.int32, (R8, 8, C), 0)
    a1 = lax.broadcasted_iota(jnp.int32, (R8, 8, C), 1)
    ln3 = lax.broadcasted_iota(jnp.int32, (R8, 8, C), 2)
    idx3 = ln3 * R + a0 * 8 + a1

    def bit0(s):
        if s >= R:
            return (ln3 & (s // R)) == 0
        if s >= 8:
            return (a0 & (s // 8)) == 0
        return (a1 & s) == 0

    z = z.reshape(R8, 8, C)
    z = z ^ -((idx3 >> 1) & 1)
    for k in range(1, LOGN + 1):
        for j in range(k - 1, -1, -1):
            s = 1 << j
            if s >= 8 and s < R:
                u = s // 8
                g = R8 // (2 * u)
                v = z.reshape(g, 2, u, 8, C)
                lo = jnp.minimum(v[:, 0], v[:, 1])
                hi = jnp.maximum(v[:, 0], v[:, 1])
                z = jnp.stack([lo, hi], axis=1).reshape(R8, 8, C)
            else:
                bs = bit0(s)
                if s >= R:
                    u, ax, wid = s // R, 2, C
                else:
                    u, ax, wid = s, 1, 8
                w_dn = _roll3(z, wid - u, ax)
                w_up = _roll3(z, u, ax)
                z = jnp.where(bs, jnp.minimum(z, w_dn),
                              jnp.maximum(z, w_up))
        if k < LOGN:
            z = z ^ -(((idx3 >> k) ^ (idx3 >> (k + 1))) & 1)
    z = z.reshape(R, C)

    y_s = ~z
    bits_s = y_s ^ ((y_s >> 31) & jnp.int32(0x7FFFFFFF))
    t_s = (bits_s & 1).astype(jnp.float32)
    e_s = lax.bitcast_convert_type(bits_s, jnp.float32)

    acc = t_s
    sh = 1
    while sh < R:
        shifted = jnp.concatenate(
            [jnp.zeros((sh, C), jnp.float32), acc[:-sh, :]], axis=0)
        acc = acc + shifted
        sh *= 2
    colcum = acc
    tot = colcum[R - 1:R, :]
    ia = lax.broadcasted_iota(jnp.int32, (C, C), 0)
    ib = lax.broadcasted_iota(jnp.int32, (C, C), 1)
    tri = (ia < ib).astype(jnp.float32)
    excl = jnp.dot(tot, tri, preferred_element_type=jnp.float32)
    cum_t = colcum + excl

    gts = jnp.sum(t_s)
    cnt = (lane * R + row + 1).astype(jnp.float32)
    cum1 = cnt - cum_t
    inter = gts - cum_t
    union = gts + cum1
    jacc = 1.0 - inter / jnp.maximum(union, 1e-6)
    wrapped = _roll(jacc, 1, 0)
    lastrow = jacc[R - 1:R, :]
    lastrow_sh = jnp.concatenate(
        [jnp.zeros((1, 1), jnp.float32), lastrow[:, :-1]], axis=1)
    prev = jnp.where(row == 0, lastrow_sh, wrapped)
    grad = jacc - prev
    loss = jnp.sum(jnp.maximum(e_s, 0.0) * grad)
    o_ref[0, 0, :] = jnp.broadcast_to(loss, (C,))


def _run(pred, target, interpret=False):
    B = pred.shape[0]
    C = 128
    R = (pred.shape[1] * pred.shape[2]) // C
    p = pred.reshape(B, R, C)
    t = target.reshape(B, R, C)
    losses = pl.pallas_call(
        _lovasz_body,
        grid=(B,),
        in_specs=[
            pl.BlockSpec((1, R, C), lambda i: (i, 0, 0)),
            pl.BlockSpec((1, R, C), lambda i: (i, 0, 0)),
        ],
        out_specs=pl.BlockSpec((1, 1, C), lambda i: (i, 0, 0)),
        out_shape=jax.ShapeDtypeStruct((B, 1, C), jnp.float32),
        interpret=interpret,
    )(p, t)
    total = jnp.sum(losses[:, 0, 0]) / B
    return jnp.where(jnp.isfinite(total), total, jnp.zeros((), jnp.float32))


def kernel(pred, target):
    return _run(pred, target)


# speedup vs baseline: 1.3155x; 1.3155x over previous
import jax
import jax.numpy as jnp
from jax import lax
from jax.experimental import pallas as pl


def _roll3(x, sh, axis):
    hi = [slice(None)] * 3
    lo = [slice(None)] * 3
    hi[axis] = slice(-sh, None)
    lo[axis] = slice(None, -sh)
    return jnp.concatenate([x[tuple(hi)], x[tuple(lo)]], axis=axis)


def _lovasz_body(p_ref, t_ref, o_ref):
    B, R, C = p_ref.shape
    N = R * C
    LOGN = N.bit_length() - 1
    R8 = R // 8
    G = B * R8

    p = p_ref[...].reshape(G, 8, C)
    t = t_ref[...].reshape(G, 8, C)

    a0 = lax.broadcasted_iota(jnp.int32, (G, 8, C), 0)
    a1 = lax.broadcasted_iota(jnp.int32, (G, 8, C), 1)
    ln3 = lax.broadcasted_iota(jnp.int32, (G, 8, C), 2)
    a0w = a0 & (R8 - 1)
    idx3 = ln3 * R + a0w * 8 + a1

    signs = 2.0 * t - 1.0
    e = 1.0 - p * signs
    bits = lax.bitcast_convert_type(e, jnp.int32)
    bits = (bits & jnp.int32(-2)) | t.astype(jnp.int32)
    y = bits ^ ((bits >> 31) & jnp.int32(0x7FFFFFFF))
    z = ~y

    def bit0(s):
        if s >= R:
            return (ln3 & (s // R)) == 0
        return (a1 & s) == 0

    z = z ^ -((idx3 >> 1) & 1)
    for k in range(1, LOGN + 1):
        for j in range(k - 1, -1, -1):
            s = 1 << j
            if 8 <= s < R:
                u = s // 8
                g = G // (2 * u)
                v = z.reshape(g, 2, u, 8, C)
                lo = jnp.minimum(v[:, 0], v[:, 1])
                hi = jnp.maximum(v[:, 0], v[:, 1])
                z = jnp.stack([lo, hi], axis=1).reshape(G, 8, C)
            else:
                bs = bit0(s)
                if s >= R:
                    u, ax, wid = s // R, 2, C
                else:
                    u, ax, wid = s, 1, 8
                w_dn = _roll3(z, wid - u, ax)
                w_up = _roll3(z, u, ax)
                z = jnp.where(bs, jnp.minimum(z, w_dn),
                              jnp.maximum(z, w_up))
        if k < LOGN:
            z = z ^ -(((idx3 >> k) ^ (idx3 >> (k + 1))) & 1)

    y_s = ~z
    bits_s = y_s ^ ((y_s >> 31) & jnp.int32(0x7FFFFFFF))
    t_s = ((bits_s & 1).astype(jnp.float32)).reshape(B, R, C)
    e_s = lax.bitcast_convert_type(bits_s, jnp.float32).reshape(B, R, C)

    acc = t_s
    sh = 1
    while sh < R:
        shifted = jnp.concatenate(
            [jnp.zeros((B, sh, C), jnp.float32), acc[:, :-sh, :]], axis=1)
        acc = acc + shifted
        sh *= 2
    colcum = acc
    tot = colcum[:, R - 1, :]
    ia = lax.broadcasted_iota(jnp.int32, (C, C), 0)
    ib = lax.broadcasted_iota(jnp.int32, (C, C), 1)
    tri = (ia < ib).astype(jnp.float32)
    excl = jnp.dot(tot, tri, preferred_element_type=jnp.float32)
    cum_t = colcum + excl[:, None, :]

    gts = jnp.sum(t_s, axis=(1, 2), keepdims=True)
    row = lax.broadcasted_iota(jnp.int32, (B, R, C), 1)
    lane = lax.broadcasted_iota(jnp.int32, (B, R, C), 2)
    cnt = (lane * R + row + 1).astype(jnp.float32)
    cum1 = cnt - cum_t
    inter = gts - cum_t
    union = gts + cum1
    jacc = 1.0 - inter / jnp.maximum(union, 1e-6)
    wrapped = jnp.concatenate([jacc[:, -1:, :], jacc[:, :-1, :]], axis=1)
    lastrow = jacc[:, R - 1:R, :]
    lastrow_sh = jnp.concatenate(
        [jnp.zeros((B, 1, 1), jnp.float32), lastrow[:, :, :-1]], axis=2)
    prev = jnp.where(row == 0, lastrow_sh, wrapped)
    grad = jacc - prev
    loss = jnp.sum(jnp.maximum(e_s, 0.0) * grad, axis=(1, 2))
    o_ref[:, 0, :] = jnp.broadcast_to(loss[:, None], (B, C))


def _run(pred, target, interpret=False):
    B = pred.shape[0]
    C = 128
    R = (pred.shape[1] * pred.shape[2]) // C
    p = pred.reshape(B, R, C)
    t = target.reshape(B, R, C)
    losses = pl.pallas_call(
        _lovasz_body,
        out_shape=jax.ShapeDtypeStruct((B, 1, C), jnp.float32),
        interpret=interpret,
    )(p, t)
    total = jnp.sum(losses[:, 0, 0]) / B
    return jnp.where(jnp.isfinite(total), total, jnp.zeros((), jnp.float32))


def kernel(pred, target):
    return _run(pred, target)


# speedup vs baseline: 1.3164x; 1.0006x over previous
import jax
import jax.numpy as jnp
from jax import lax
from jax.experimental import pallas as pl


def _roll3(x, sh, axis):
    hi = [slice(None)] * 3
    lo = [slice(None)] * 3
    hi[axis] = slice(-sh, None)
    lo[axis] = slice(None, -sh)
    return jnp.concatenate([x[tuple(hi)], x[tuple(lo)]], axis=axis)


def _lovasz_body(p_ref, t_ref, o_ref):
    B, R, C = p_ref.shape
    N = R * C
    LOGN = N.bit_length() - 1
    R8 = R // 8
    G = B * R8

    p = p_ref[...].reshape(G, 8, C)
    t = t_ref[...].reshape(G, 8, C)

    a0 = lax.broadcasted_iota(jnp.int32, (G, 8, C), 0)
    a1 = lax.broadcasted_iota(jnp.int32, (G, 8, C), 1)
    ln3 = lax.broadcasted_iota(jnp.int32, (G, 8, C), 2)
    a0w = a0 & (R8 - 1)
    idx3 = ln3 * R + a0w * 8 + a1

    signs = 2.0 * t - 1.0
    e = 1.0 - p * signs
    bits = lax.bitcast_convert_type(e, jnp.int32)
    bits = (bits & jnp.int32(-2)) | t.astype(jnp.int32)
    y = bits ^ ((bits >> 31) & jnp.int32(0x7FFFFFFF))
    z = ~y

    def bit0(s):
        if s >= R:
            return (ln3 & (s // R)) == 0
        return (a1 & s) == 0

    z = z ^ -((idx3 >> 1) & 1)
    for k in range(1, LOGN + 1):
        js = list(range(k - 1, -1, -1))
        i = 0
        while i < len(js):
            s = 1 << js[i]
            if 8 <= s < R:
                u = s // 8
                if i + 1 < len(js) and 8 <= (1 << js[i + 1]) < R:
                    u2 = u // 2
                    g = G // (2 * u)
                    v = z.reshape(g, 2, 2, u2, 8, C)
                    a0_ = jnp.minimum(v[:, 0], v[:, 1])
                    a1_ = jnp.maximum(v[:, 0], v[:, 1])
                    q00 = jnp.minimum(a0_[:, 0], a0_[:, 1])
                    q01 = jnp.maximum(a0_[:, 0], a0_[:, 1])
                    q10 = jnp.minimum(a1_[:, 0], a1_[:, 1])
                    q11 = jnp.maximum(a1_[:, 0], a1_[:, 1])
                    z = jnp.stack([
                        jnp.stack([q00, q01], axis=1),
                        jnp.stack([q10, q11], axis=1)], axis=1)
                    z = z.reshape(G, 8, C)
                    i += 2
                    continue
                g = G // (2 * u)
                v = z.reshape(g, 2, u, 8, C)
                lo = jnp.minimum(v[:, 0], v[:, 1])
                hi = jnp.maximum(v[:, 0], v[:, 1])
                z = jnp.stack([lo, hi], axis=1).reshape(G, 8, C)
            else:
                bs = bit0(s)
                if s >= R:
                    u, ax, wid = s // R, 2, C
                else:
                    u, ax, wid = s, 1, 8
                w_dn = _roll3(z, wid - u, ax)
                w_up = _roll3(z, u, ax)
                z = jnp.where(bs, jnp.minimum(z, w_dn),
                              jnp.maximum(z, w_up))
            i += 1
        if k < LOGN:
            z = z ^ -(((idx3 >> k) ^ (idx3 >> (k + 1))) & 1)

    y_s = ~z
    bits_s = y_s ^ ((y_s >> 31) & jnp.int32(0x7FFFFFFF))
    t_s = ((bits_s & 1).astype(jnp.float32)).reshape(B, R, C)
    e_s = lax.bitcast_convert_type(bits_s, jnp.float32).reshape(B, R, C)

    acc = t_s
    sh = 1
    while sh < R:
        shifted = jnp.concatenate(
            [jnp.zeros((B, sh, C), jnp.float32), acc[:, :-sh, :]], axis=1)
        acc = acc + shifted
        sh *= 2
    colcum = acc
    tot = colcum[:, R - 1, :]
    ia = lax.broadcasted_iota(jnp.int32, (C, C), 0)
    ib = lax.broadcasted_iota(jnp.int32, (C, C), 1)
    tri = (ia < ib).astype(jnp.float32)
    excl = jnp.dot(tot, tri, preferred_element_type=jnp.float32)
    cum_t = colcum + excl[:, None, :]

    gts = jnp.sum(t_s, axis=(1, 2), keepdims=True)
    row = lax.broadcasted_iota(jnp.int32, (B, R, C), 1)
    lane = lax.broadcasted_iota(jnp.int32, (B, R, C), 2)
    cnt = (lane * R + row + 1).astype(jnp.float32)
    cum1 = cnt - cum_t
    inter = gts - cum_t
    union = gts + cum1
    jacc = 1.0 - inter / jnp.maximum(union, 1e-6)
    wrapped = jnp.concatenate([jacc[:, -1:, :], jacc[:, :-1, :]], axis=1)
    lastrow = jacc[:, R - 1:R, :]
    lastrow_sh = jnp.concatenate(
        [jnp.zeros((B, 1, 1), jnp.float32), lastrow[:, :, :-1]], axis=2)
    prev = jnp.where(row == 0, lastrow_sh, wrapped)
    grad = jacc - prev
    loss = jnp.sum(jnp.maximum(e_s, 0.0) * grad, axis=(1, 2))
    o_ref[:, 0, :] = jnp.broadcast_to(loss[:, None], (B, C))


def _run(pred, target, interpret=False):
    B = pred.shape[0]
    C = 128
    R = (pred.shape[1] * pred.shape[2]) // C
    p = pred.reshape(B, R, C)
    t = target.reshape(B, R, C)
    losses = pl.pallas_call(
        _lovasz_body,
        out_shape=jax.ShapeDtypeStruct((B, 1, C), jnp.float32),
        interpret=interpret,
    )(p, t)
    total = jnp.sum(losses[:, 0, 0]) / B
    return jnp.where(jnp.isfinite(total), total, jnp.zeros((), jnp.float32))


def kernel(pred, target):
    return _run(pred, target)
